# baseline (speedup 1.0000x reference)
import functools

import jax
import jax.numpy as jnp
from jax import lax
from jax.experimental import pallas as pl
from jax.experimental.pallas import tpu as pltpu
from jax.experimental.pallas import tpu_sc as plsc

B = 16384
D = 64
NC = 2
NS = 16
NW = NC * NS
BPW = B // NW
CHUNK = 128
NCHUNK = BPW // CHUNK


def _body(users_hbm, pos_hbm, neg_hbm, utab_hbm, itab_hbm,
          out_p_hbm, out_n_hbm,
          idx_u, idx_p, idx_n, urows, prows, nrows, outp_v, outn_v, sem):
    wid = lax.axis_index("s") * NC + lax.axis_index("c")

    pltpu.sync_copy(users_hbm.at[wid], idx_u)
    pltpu.sync_copy(pos_hbm.at[wid], idx_p)
    pltpu.sync_copy(neg_hbm.at[wid], idx_n)

    handles = []
    for j in range(NCHUNK):
        dst = pl.ds(j * CHUNK, CHUNK)
        handles.append(pltpu.async_copy(utab_hbm.at[idx_u.at[j]], urows.at[dst], sem))
        handles.append(pltpu.async_copy(itab_hbm.at[idx_p.at[j]], prows.at[dst], sem))
        handles.append(pltpu.async_copy(itab_hbm.at[idx_n.at[j]], nrows.at[dst], sem))
    for h in handles:
        h.wait()

    lane = lax.iota(jnp.int32, 16)
    cols = [jnp.full((16,), j, jnp.int32) for j in range(D)]

    def group(g, carry):
        row_idx = g * 16 + lane
        accp = jnp.zeros((16,), jnp.float32)
        accn = jnp.zeros((16,), jnp.float32)
        for j in range(D):
            u_c = plsc.load_gather(urows, [row_idx, cols[j]])
            p_c = plsc.load_gather(prows, [row_idx, cols[j]])
            n_c = plsc.load_gather(nrows, [row_idx, cols[j]])
            accp = accp + u_c * p_c
            accn = accn + u_c * n_c
        outp_v[pl.ds(g * 16, 16)] = accp
        outn_v[pl.ds(g * 16, 16)] = accn
        return carry

    lax.fori_loop(0, BPW // 16, group, 0)

    pltpu.sync_copy(outp_v, out_p_hbm.at[wid])
    pltpu.sync_copy(outn_v, out_n_hbm.at[wid])


@jax.jit
def kernel(users, positives, negatives, user_table, item_table):
    users = users.astype(jnp.int32).reshape(NW, NCHUNK, CHUNK)
    positives = positives.astype(jnp.int32).reshape(NW, NCHUNK, CHUNK)
    negatives = negatives.astype(jnp.int32).reshape(NW, NCHUNK, CHUNK)

    mesh = plsc.VectorSubcoreMesh(core_axis_name="c", subcore_axis_name="s")
    f = pl.kernel(
        _body,
        mesh=mesh,
        compiler_params=pltpu.CompilerParams(
            needs_layout_passes=False, use_tc_tiling_on_sc=False),
        out_type=(
            jax.ShapeDtypeStruct((NW, BPW), jnp.float32),
            jax.ShapeDtypeStruct((NW, BPW), jnp.float32),
        ),
        scratch_types=[
            pltpu.VMEM((NCHUNK, CHUNK), jnp.int32),
            pltpu.VMEM((NCHUNK, CHUNK), jnp.int32),
            pltpu.VMEM((NCHUNK, CHUNK), jnp.int32),
            pltpu.VMEM((BPW, D), jnp.float32),
            pltpu.VMEM((BPW, D), jnp.float32),
            pltpu.VMEM((BPW, D), jnp.float32),
            pltpu.VMEM((BPW,), jnp.float32),
            pltpu.VMEM((BPW,), jnp.float32),
            pltpu.SemaphoreType.DMA,
        ],
    )
    out_p, out_n = f(users, positives, negatives, user_table, item_table)
    return out_p.reshape(B), out_n.reshape(B)


# speedup vs baseline: 2.0745x; 2.0745x over previous
import functools

import jax
import jax.numpy as jnp
from jax import lax
from jax.experimental import pallas as pl
from jax.experimental.pallas import tpu as pltpu
from jax.experimental.pallas import tpu_sc as plsc

B = 16384
D = 64
NC = 2
NS = 16
NW = NC * NS
BPW = B // NW
CHUNK = 32
NCHUNK = BPW // CHUNK


def _bfly(v):
    lane = lax.iota(jnp.int32, 16)
    for k in (1, 2, 4, 8):
        v = v + v.at[lane ^ k].get(mode="promise_in_bounds")
    return v


def _body(users_hbm, pos_hbm, neg_hbm, utab_hbm, itab_hbm,
          out_p_hbm, out_n_hbm,
          idx_u, idx_p, idx_n, gidx_u, gidx_p, gidx_n,
          ubuf, pbuf, nbuf, outp_v, outn_v, sem):
    wid = lax.axis_index("s") * NC + lax.axis_index("c")

    pltpu.sync_copy(users_hbm.at[wid], idx_u)
    pltpu.sync_copy(pos_hbm.at[wid], idx_p)
    pltpu.sync_copy(neg_hbm.at[wid], idx_n)

    def shift(i, carry):
        s = pl.ds(i * 16, 16)
        gidx_u[s] = lax.shift_right_logical(idx_u[s], 3)
        gidx_p[s] = lax.shift_right_logical(idx_p[s], 3)
        gidx_n[s] = lax.shift_right_logical(idx_n[s], 3)
        return carry

    lax.fori_loop(0, BPW // 16, shift, 0)

    lane = lax.iota(jnp.int32, 16)

    def chunk(g, carry):
        base = g * CHUNK
        gvu = gidx_u[pl.ds(base, 16)]
        gvp = gidx_p[pl.ds(base, 16)]
        gvn = gidx_n[pl.ds(base, 16)]
        gvu2 = gidx_u[pl.ds(base + 16, 16)]
        gvp2 = gidx_p[pl.ds(base + 16, 16)]
        gvn2 = gidx_n[pl.ds(base + 16, 16)]
        handles = []
        for c in range(CHUNK):
            tu = (gvu if c < 16 else gvu2)[c % 16]
            tp = (gvp if c < 16 else gvp2)[c % 16]
            tn = (gvn if c < 16 else gvn2)[c % 16]
            handles.append(pltpu.async_copy(utab_hbm.at[tu], ubuf.at[c], sem))
            handles.append(pltpu.async_copy(itab_hbm.at[tp], pbuf.at[c], sem))
            handles.append(pltpu.async_copy(itab_hbm.at[tn], nbuf.at[c], sem))
        for h in handles:
            h.wait()

        for h in range(CHUNK // 16):
            svu = idx_u[pl.ds(base + h * 16, 16)] & 7
            svp = idx_p[pl.ds(base + h * 16, 16)] & 7
            svn = idx_n[pl.ds(base + h * 16, 16)] & 7
            resp = jnp.zeros((16,), jnp.float32)
            resn = jnp.zeros((16,), jnp.float32)
            for t in range(16):
                c = h * 16 + t
                su = svu[t]
                sp = svp[t]
                sn = svn[t]
                accp = None
                accn = None
                for k in range(D // 16):
                    u_k = ubuf[c, su, pl.ds(16 * k, 16)]
                    p_k = pbuf[c, sp, pl.ds(16 * k, 16)]
                    n_k = nbuf[c, sn, pl.ds(16 * k, 16)]
                    accp = u_k * p_k if accp is None else accp + u_k * p_k
                    accn = u_k * n_k if accn is None else accn + u_k * n_k
                resp = jnp.where(lane == t, _bfly(accp), resp)
                resn = jnp.where(lane == t, _bfly(accn), resn)
            outp_v[pl.ds(base + h * 16, 16)] = resp
            outn_v[pl.ds(base + h * 16, 16)] = resn
        return carry

    lax.fori_loop(0, NCHUNK, chunk, 0)

    pltpu.sync_copy(outp_v, out_p_hbm.at[wid])
    pltpu.sync_copy(outn_v, out_n_hbm.at[wid])


@jax.jit
def kernel(users, positives, negatives, user_table, item_table):
    users = users.astype(jnp.int32).reshape(NW, BPW)
    positives = positives.astype(jnp.int32).reshape(NW, BPW)
    negatives = negatives.astype(jnp.int32).reshape(NW, BPW)
    utab = user_table.reshape(user_table.shape[0] // 8, 8, D)
    itab = item_table.reshape(item_table.shape[0] // 8, 8, D)

    mesh = plsc.VectorSubcoreMesh(core_axis_name="c", subcore_axis_name="s")
    f = pl.kernel(
        _body,
        mesh=mesh,
        compiler_params=pltpu.CompilerParams(
            needs_layout_passes=False, use_tc_tiling_on_sc=True),
        out_type=(
            jax.ShapeDtypeStruct((NW, BPW), jnp.float32),
            jax.ShapeDtypeStruct((NW, BPW), jnp.float32),
        ),
        scratch_types=[
            pltpu.VMEM((BPW,), jnp.int32),
            pltpu.VMEM((BPW,), jnp.int32),
            pltpu.VMEM((BPW,), jnp.int32),
            pltpu.VMEM((BPW,), jnp.int32),
            pltpu.VMEM((BPW,), jnp.int32),
            pltpu.VMEM((BPW,), jnp.int32),
            pltpu.VMEM((CHUNK, 8, D), jnp.float32),
            pltpu.VMEM((CHUNK, 8, D), jnp.float32),
            pltpu.VMEM((CHUNK, 8, D), jnp.float32),
            pltpu.VMEM((BPW,), jnp.float32),
            pltpu.VMEM((BPW,), jnp.float32),
            pltpu.SemaphoreType.DMA,
        ],
    )
    out_p, out_n = f(users, positives, negatives, utab, itab)
    return out_p.reshape(B), out_n.reshape(B)
